# Initial kernel scaffold; baseline (speedup 1.0000x reference)
import jax
import jax.numpy as jnp
from jax.experimental import pallas as pl


def kernel(drug_pairs, drug_targets, conc, W1, b1, W2, b2, W3, b3):
    raise NotImplementedError("write your pallas kernel here")


# speedup vs baseline: 2.7528x; 2.7528x over previous
import functools

import jax
import jax.numpy as jnp
from jax.experimental import pallas as pl
from jax.experimental.pallas import tpu as pltpu


def _k1_gather_sum(tgt_s, w1_hbm, out_ref, land, sem, *, T, DB, off_b):
    blk = pl.program_id(0)
    base = blk * DB * T
    for di in range(DB):
        for t in range(T):
            r = tgt_s[base + di * T + t]
            pltpu.make_async_copy(w1_hbm.at[r], land.at[di, 0, t], sem).start()
            pltpu.make_async_copy(w1_hbm.at[r + off_b], land.at[di, 1, t], sem).start()
    pltpu.make_async_copy(land, land, sem).wait()
    for di in range(DB):
        for h in range(2):
            acc = land[di, h, 0]
            for t in range(1, T):
                acc = acc + land[di, h, t]
            out_ref[di, h] = acc


def _k2_combine(dp_s, cadj_ref, a_ref, b_ref, b1_ref, sab_hbm, out_ref,
                land1, land2, sem1, sem2, *, BS):
    blk = pl.program_id(0)
    base = blk * BS * 2
    for i in range(BS):
        d1 = dp_s[base + 2 * i]
        d2 = dp_s[base + 2 * i + 1]
        pltpu.make_async_copy(sab_hbm.at[d1], land1.at[i], sem1).start()
        pltpu.make_async_copy(sab_hbm.at[d2], land2.at[i], sem2).start()
    pltpu.make_async_copy(land1, land1, sem1).wait()
    pltpu.make_async_copy(land2, land2, sem2).wait()
    c1 = cadj_ref[:, 0:1].reshape(BS, 1, 1)
    c2 = cadj_ref[:, 1:2].reshape(BS, 1, 1)
    a = a_ref[...]
    b = b_ref[...]
    b1 = b1_ref[...]
    x1 = land1[:, 0] + land2[:, 1] + c1 * a + c2 * b + b1
    x2 = land2[:, 0] + land1[:, 1] + c2 * a + c1 * b + b1
    out_ref[0] = jnp.maximum(x1, 0.0).astype(jnp.bfloat16)
    out_ref[1] = jnp.maximum(x2, 0.0).astype(jnp.bfloat16)


def _k3_mlp(x_ref, w2_ref, b2_ref, w3_ref, b3_ref, o_ref):
    h = jnp.dot(x_ref[...], w2_ref[...], preferred_element_type=jnp.float32)
    h = jnp.maximum(h + b2_ref[...], 0.0).astype(jnp.bfloat16)
    y = jnp.dot(h, w3_ref[...], preferred_element_type=jnp.float32)
    o_ref[...] = y + b3_ref[...]


def _pick_block(n, want):
    for cand in (want, 256, 128, 64, 32, 16, 8, 4, 2, 1):
        if cand <= want and n % cand == 0:
            return cand
    return 1


def kernel(drug_pairs, drug_targets, conc, W1, b1, W2, b2, W3, b3):
    B = drug_pairs.shape[0]
    D, T = drug_targets.shape
    DIN, H1 = W1.shape
    H2 = W2.shape[1]
    P = (DIN - 2) // 2
    S = H1 // 128

    tgt = drug_targets.astype(jnp.int32)
    eq = tgt[:, :, None] == tgt[:, None, :]
    earlier = jnp.tril(jnp.ones((T, T), jnp.bool_), k=-1)
    isdup = jnp.any(eq & earlier[None], axis=2)
    tgt_a = jnp.where(isdup, P, tgt)
    ndup = jnp.sum(isdup, axis=1).astype(jnp.float32)
    dp = drug_pairs.astype(jnp.int32)
    cadj = conc.astype(jnp.float32) - ndup[dp]

    w1r = W1.reshape(DIN, S, 128)
    a3 = W1[P].reshape(1, S, 128)
    brow = W1[2 * P + 1].reshape(1, S, 128)
    b1r = b1.reshape(1, S, 128)
    tgt_flat = tgt_a.reshape(D * T)
    dp_flat = dp.reshape(2 * B)

    DB = _pick_block(D, 8)
    BS = _pick_block(B, 256)
    BS3 = _pick_block(2 * B, 512)

    sab = pl.pallas_call(
        functools.partial(_k1_gather_sum, T=T, DB=DB, off_b=P + 1),
        grid=(D // DB,),
        in_specs=[
            pl.BlockSpec(memory_space=pltpu.SMEM),
            pl.BlockSpec(memory_space=pl.ANY),
        ],
        out_specs=pl.BlockSpec((DB, 2, S, 128), lambda i: (i, 0, 0, 0)),
        out_shape=jax.ShapeDtypeStruct((D, 2, S, 128), jnp.float32),
        scratch_shapes=[
            pltpu.VMEM((DB, 2, T, S, 128), jnp.float32),
            pltpu.SemaphoreType.DMA,
        ],
        compiler_params=pltpu.CompilerParams(
            dimension_semantics=("parallel",)),
        name="drug_gather_sum",
    )(tgt_flat, w1r)

    xh = pl.pallas_call(
        functools.partial(_k2_combine, BS=BS),
        grid=(B // BS,),
        in_specs=[
            pl.BlockSpec(memory_space=pltpu.SMEM),
            pl.BlockSpec((BS, 2), lambda i: (i, 0)),
            pl.BlockSpec((1, S, 128), lambda i: (0, 0, 0)),
            pl.BlockSpec((1, S, 128), lambda i: (0, 0, 0)),
            pl.BlockSpec((1, S, 128), lambda i: (0, 0, 0)),
            pl.BlockSpec(memory_space=pl.ANY),
        ],
        out_specs=pl.BlockSpec((2, BS, S, 128), lambda i: (0, i, 0, 0)),
        out_shape=jax.ShapeDtypeStruct((2, B, S, 128), jnp.bfloat16),
        scratch_shapes=[
            pltpu.VMEM((BS, 2, S, 128), jnp.float32),
            pltpu.VMEM((BS, 2, S, 128), jnp.float32),
            pltpu.SemaphoreType.DMA,
            pltpu.SemaphoreType.DMA,
        ],
        compiler_params=pltpu.CompilerParams(
            dimension_semantics=("parallel",)),
        name="combine_relu",
    )(dp_flat, cadj, a3, brow, b1r, sab)

    xall = xh.reshape(2 * B, H1)
    w2b = W2.astype(jnp.bfloat16)
    b2r = b2.reshape(1, H2)
    w3p = jnp.pad(W3, ((0, 0), (0, 127))).astype(jnp.bfloat16)
    b3p = jnp.pad(b3.reshape(1, 1), ((0, 0), (0, 127)))

    y2 = pl.pallas_call(
        _k3_mlp,
        grid=(2 * B // BS3,),
        in_specs=[
            pl.BlockSpec((BS3, H1), lambda i: (i, 0)),
            pl.BlockSpec((H1, H2), lambda i: (0, 0)),
            pl.BlockSpec((1, H2), lambda i: (0, 0)),
            pl.BlockSpec((H2, 128), lambda i: (0, 0)),
            pl.BlockSpec((1, 128), lambda i: (0, 0)),
        ],
        out_specs=pl.BlockSpec((BS3, 128), lambda i: (i, 0)),
        out_shape=jax.ShapeDtypeStruct((2 * B, 128), jnp.float32),
        compiler_params=pltpu.CompilerParams(
            dimension_semantics=("parallel",)),
        name="mlp_tail",
    )(xall, w2b, b2r, w3p, b3p)

    return (y2[0:B, 0], y2[B:2 * B, 0])


# speedup vs baseline: 3.2315x; 1.1739x over previous
import functools

import jax
import jax.numpy as jnp
from jax.experimental import pallas as pl
from jax.experimental.pallas import tpu as pltpu


def _k1_gather_sum(tgt_s, wcat_hbm, out_ref, land, sem, *, T, DB, NBI):
    blk = pl.program_id(0) * NBI + pl.program_id(1)
    base = blk * DB * T
    for di in range(DB):
        for t in range(T):
            r = tgt_s[base + di * T + t]
            pltpu.make_async_copy(wcat_hbm.at[r], land.at[di, t], sem).start()
    pltpu.make_async_copy(land, land, sem).wait()
    for di in range(DB):
        acc = land[di, 0].astype(jnp.float32)
        for t in range(1, T):
            acc = acc + land[di, t].astype(jnp.float32)
        out_ref[di] = acc


def _k2_combine(dp_s, cadj_ref, a_ref, b_ref, b1_ref, sab_hbm, out_ref,
                land1, land2, sem1, sem2, *, BS, S, NBI):
    blk = pl.program_id(0) * NBI + pl.program_id(1)
    base = blk * BS * 2
    for i in range(BS):
        d1 = dp_s[base + 2 * i]
        d2 = dp_s[base + 2 * i + 1]
        pltpu.make_async_copy(sab_hbm.at[d1], land1.at[i], sem1).start()
        pltpu.make_async_copy(sab_hbm.at[d2], land2.at[i], sem2).start()
    pltpu.make_async_copy(land1, land1, sem1).wait()
    pltpu.make_async_copy(land2, land2, sem2).wait()
    c1 = cadj_ref[:, 0:1].reshape(BS, 1, 1)
    c2 = cadj_ref[:, 1:2].reshape(BS, 1, 1)
    a = a_ref[...]
    b = b_ref[...]
    b1 = b1_ref[...]
    x1 = land1[:, :S] + land2[:, S:] + c1 * a + c2 * b + b1
    x2 = land2[:, :S] + land1[:, S:] + c2 * a + c1 * b + b1
    out_ref[0] = jnp.maximum(x1, 0.0).astype(jnp.bfloat16)
    out_ref[1] = jnp.maximum(x2, 0.0).astype(jnp.bfloat16)


def _k3_mlp(x_ref, w2_ref, b2_ref, w3_ref, b3_ref, o_ref):
    h = jnp.dot(x_ref[...], w2_ref[...], preferred_element_type=jnp.float32)
    h = jnp.maximum(h + b2_ref[...], 0.0).astype(jnp.bfloat16)
    y = jnp.dot(h, w3_ref[...], preferred_element_type=jnp.float32)
    o_ref[...] = y + b3_ref[...]


def _pick_block(n, want):
    for cand in (want, 256, 128, 64, 32, 16, 8, 4, 2, 1):
        if cand <= want and n % cand == 0:
            return cand
    return 1


def kernel(drug_pairs, drug_targets, conc, W1, b1, W2, b2, W3, b3):
    B = drug_pairs.shape[0]
    D, T = drug_targets.shape
    DIN, H1 = W1.shape
    H2 = W2.shape[1]
    P = (DIN - 2) // 2
    S = H1 // 128

    tgt = drug_targets.astype(jnp.int32)
    eq = tgt[:, :, None] == tgt[:, None, :]
    earlier = jnp.tril(jnp.ones((T, T), jnp.bool_), k=-1)
    isdup = jnp.any(eq & earlier[None], axis=2)
    tgt_a = jnp.where(isdup, P, tgt)
    ndup = jnp.sum(isdup, axis=1).astype(jnp.float32)
    dp = drug_pairs.astype(jnp.int32)
    cadj = conc.astype(jnp.float32) - ndup[dp]

    wcat = jnp.concatenate([W1[:P], W1[P + 1:2 * P + 1]], axis=1)
    ab = jnp.concatenate([W1[P], W1[2 * P + 1]])[None]
    wcat = jnp.concatenate([wcat, ab], axis=0).astype(jnp.bfloat16)
    wcat = wcat.reshape(P + 1, 2 * S, 128)

    a3 = W1[P].reshape(1, S, 128)
    brow = W1[2 * P + 1].reshape(1, S, 128)
    b1r = b1.reshape(1, S, 128)
    tgt_flat = tgt_a.reshape(D * T)
    dp_flat = dp.reshape(2 * B)

    DB = _pick_block(D, 8)
    BS = _pick_block(B, 256)
    BS3 = _pick_block(2 * B, 512)
    NC = 1
    NB1 = D // DB // NC
    NB2 = B // BS // NC
    NB3 = 2 * B // BS3 // NC
    sem1 = ("core_parallel", "arbitrary") if NC == 2 else ("arbitrary", "arbitrary")

    sab = pl.pallas_call(
        functools.partial(_k1_gather_sum, T=T, DB=DB, NBI=NB1),
        grid=(NC, NB1),
        in_specs=[
            pl.BlockSpec(memory_space=pltpu.SMEM),
            pl.BlockSpec(memory_space=pl.ANY),
        ],
        out_specs=pl.BlockSpec((DB, 2 * S, 128),
                               lambda c, i: (c * NB1 + i, 0, 0)),
        out_shape=jax.ShapeDtypeStruct((D, 2 * S, 128), jnp.float32),
        scratch_shapes=[
            pltpu.VMEM((DB, T, 2 * S, 128), jnp.bfloat16),
            pltpu.SemaphoreType.DMA,
        ],
        compiler_params=pltpu.CompilerParams(
            dimension_semantics=sem1),
        name="drug_gather_sum",
    )(tgt_flat, wcat)

    xh = pl.pallas_call(
        functools.partial(_k2_combine, BS=BS, S=S, NBI=NB2),
        grid=(NC, NB2),
        in_specs=[
            pl.BlockSpec(memory_space=pltpu.SMEM),
            pl.BlockSpec((BS, 2), lambda c, i: (c * NB2 + i, 0)),
            pl.BlockSpec((1, S, 128), lambda c, i: (0, 0, 0)),
            pl.BlockSpec((1, S, 128), lambda c, i: (0, 0, 0)),
            pl.BlockSpec((1, S, 128), lambda c, i: (0, 0, 0)),
            pl.BlockSpec(memory_space=pl.ANY),
        ],
        out_specs=pl.BlockSpec((2, BS, S, 128),
                               lambda c, i: (0, c * NB2 + i, 0, 0)),
        out_shape=jax.ShapeDtypeStruct((2, B, S, 128), jnp.bfloat16),
        scratch_shapes=[
            pltpu.VMEM((BS, 2 * S, 128), jnp.float32),
            pltpu.VMEM((BS, 2 * S, 128), jnp.float32),
            pltpu.SemaphoreType.DMA,
            pltpu.SemaphoreType.DMA,
        ],
        compiler_params=pltpu.CompilerParams(
            dimension_semantics=sem1),
        name="combine_relu",
    )(dp_flat, cadj, a3, brow, b1r, sab)

    xall = xh.reshape(2 * B, H1)
    w2b = W2.astype(jnp.bfloat16)
    b2r = b2.reshape(1, H2)
    w3p = jnp.pad(W3, ((0, 0), (0, 127))).astype(jnp.bfloat16)
    b3p = jnp.pad(b3.reshape(1, 1), ((0, 0), (0, 127)))

    y2 = pl.pallas_call(
        _k3_mlp,
        grid=(NC, NB3),
        in_specs=[
            pl.BlockSpec((BS3, H1), lambda c, i: (c * NB3 + i, 0)),
            pl.BlockSpec((H1, H2), lambda c, i: (0, 0)),
            pl.BlockSpec((1, H2), lambda c, i: (0, 0)),
            pl.BlockSpec((H2, 128), lambda c, i: (0, 0)),
            pl.BlockSpec((1, 128), lambda c, i: (0, 0)),
        ],
        out_specs=pl.BlockSpec((BS3, 128), lambda c, i: (c * NB3 + i, 0)),
        out_shape=jax.ShapeDtypeStruct((2 * B, 128), jnp.float32),
        compiler_params=pltpu.CompilerParams(
            dimension_semantics=sem1),
        name="mlp_tail",
    )(xall, w2b, b2r, w3p, b3p)

    return (y2[0:B, 0], y2[B:2 * B, 0])


# speedup vs baseline: 4.0507x; 1.2535x over previous
import functools

import jax
import jax.numpy as jnp
from jax.experimental import pallas as pl
from jax.experimental.pallas import tpu as pltpu


def _k1_gather_sum(tgt_s, wcat_hbm, out_ref, land, sems, *, T, DB, NBI):
    j = pl.program_id(1)
    blk = pl.program_id(0) * NBI + j
    cur = jax.lax.rem(j, 2)
    nxt = 1 - cur

    def issue(b, slot):
        base = b * DB * T
        for di in range(DB):
            for t in range(T):
                r = tgt_s[base + di * T + t]
                pltpu.make_async_copy(
                    wcat_hbm.at[r], land.at[slot, di, t], sems.at[slot]).start()

    @pl.when(j == 0)
    def _():
        issue(blk, cur)

    pltpu.make_async_copy(land.at[cur], land.at[cur], sems.at[cur]).wait()

    @pl.when(j + 1 < NBI)
    def _():
        issue(blk + 1, nxt)

    for di in range(DB):
        acc = land[cur, di, 0].astype(jnp.float32)
        for t in range(1, T):
            acc = acc + land[cur, di, t].astype(jnp.float32)
        out_ref[di] = acc


def _k2_combine(dp_s, cadj_ref, a_ref, b_ref, b1_ref, sab_hbm, out_ref,
                land1, land2, sems, *, BS, S, NBI):
    j = pl.program_id(1)
    blk = pl.program_id(0) * NBI + j
    cur = jax.lax.rem(j, 2)
    nxt = 1 - cur

    def issue(b, slot):
        base = b * BS * 2
        for i in range(BS):
            d1 = dp_s[base + 2 * i]
            d2 = dp_s[base + 2 * i + 1]
            pltpu.make_async_copy(
                sab_hbm.at[d1], land1.at[slot, i], sems.at[slot]).start()
            pltpu.make_async_copy(
                sab_hbm.at[d2], land2.at[slot, i], sems.at[slot]).start()

    @pl.when(j == 0)
    def _():
        issue(blk, cur)

    pltpu.make_async_copy(land1.at[cur], land1.at[cur], sems.at[cur]).wait()
    pltpu.make_async_copy(land2.at[cur], land2.at[cur], sems.at[cur]).wait()

    @pl.when(j + 1 < NBI)
    def _():
        issue(blk + 1, nxt)

    c1 = cadj_ref[:, 0:1].reshape(BS, 1, 1)
    c2 = cadj_ref[:, 1:2].reshape(BS, 1, 1)
    a = a_ref[...]
    b = b_ref[...]
    b1 = b1_ref[...]
    l1 = land1[cur]
    l2 = land2[cur]
    x1 = l1[:, :S] + l2[:, S:] + c1 * a + c2 * b + b1
    x2 = l2[:, :S] + l1[:, S:] + c2 * a + c1 * b + b1
    out_ref[0] = jnp.maximum(x1, 0.0).astype(jnp.bfloat16)
    out_ref[1] = jnp.maximum(x2, 0.0).astype(jnp.bfloat16)


def _k3_mlp(x_ref, w2_ref, b2_ref, w3_ref, b3_ref, o_ref):
    h = jnp.dot(x_ref[...], w2_ref[...], preferred_element_type=jnp.float32)
    h = jnp.maximum(h + b2_ref[...], 0.0).astype(jnp.bfloat16)
    y = jnp.dot(h, w3_ref[...], preferred_element_type=jnp.float32)
    o_ref[...] = y + b3_ref[...]


def _pick_block(n, want):
    for cand in (want, 256, 128, 64, 32, 16, 8, 4, 2, 1):
        if cand <= want and n % cand == 0:
            return cand
    return 1


def kernel(drug_pairs, drug_targets, conc, W1, b1, W2, b2, W3, b3):
    B = drug_pairs.shape[0]
    D, T = drug_targets.shape
    DIN, H1 = W1.shape
    H2 = W2.shape[1]
    P = (DIN - 2) // 2
    S = H1 // 128

    tgt = drug_targets.astype(jnp.int32)
    eq = tgt[:, :, None] == tgt[:, None, :]
    earlier = jnp.tril(jnp.ones((T, T), jnp.bool_), k=-1)
    isdup = jnp.any(eq & earlier[None], axis=2)
    tgt_a = jnp.where(isdup, P, tgt)
    ndup = jnp.sum(isdup, axis=1).astype(jnp.float32)
    dp = drug_pairs.astype(jnp.int32)
    cadj = conc.astype(jnp.float32) - ndup[dp]

    wcat = jnp.concatenate([W1[:P], W1[P + 1:2 * P + 1]], axis=1)
    ab = jnp.concatenate([W1[P], W1[2 * P + 1]])[None]
    wcat = jnp.concatenate([wcat, ab], axis=0).astype(jnp.bfloat16)
    wcat = wcat.reshape(P + 1, 2 * S, 128)

    a3 = W1[P].reshape(1, S, 128)
    brow = W1[2 * P + 1].reshape(1, S, 128)
    b1r = b1.reshape(1, S, 128)
    tgt_flat = tgt_a.reshape(D * T)
    dp_flat = dp.reshape(2 * B)

    DB = _pick_block(D, 16)
    BS = _pick_block(B, 256)
    BS3 = _pick_block(2 * B, 512)
    NC = 1
    NB1 = D // DB // NC
    NB2 = B // BS // NC
    NB3 = 2 * B // BS3 // NC
    sem1 = ("core_parallel", "arbitrary") if NC == 2 else ("arbitrary", "arbitrary")

    sab = pl.pallas_call(
        functools.partial(_k1_gather_sum, T=T, DB=DB, NBI=NB1),
        grid=(NC, NB1),
        in_specs=[
            pl.BlockSpec(memory_space=pltpu.SMEM),
            pl.BlockSpec(memory_space=pl.ANY),
        ],
        out_specs=pl.BlockSpec((DB, 2 * S, 128),
                               lambda c, i: (c * NB1 + i, 0, 0)),
        out_shape=jax.ShapeDtypeStruct((D, 2 * S, 128), jnp.float32),
        scratch_shapes=[
            pltpu.VMEM((2, DB, T, 2 * S, 128), jnp.bfloat16),
            pltpu.SemaphoreType.DMA((2,)),
        ],
        compiler_params=pltpu.CompilerParams(
            dimension_semantics=sem1),
        name="drug_gather_sum",
    )(tgt_flat, wcat)

    xh = pl.pallas_call(
        functools.partial(_k2_combine, BS=BS, S=S, NBI=NB2),
        grid=(NC, NB2),
        in_specs=[
            pl.BlockSpec(memory_space=pltpu.SMEM),
            pl.BlockSpec((BS, 2), lambda c, i: (c * NB2 + i, 0)),
            pl.BlockSpec((1, S, 128), lambda c, i: (0, 0, 0)),
            pl.BlockSpec((1, S, 128), lambda c, i: (0, 0, 0)),
            pl.BlockSpec((1, S, 128), lambda c, i: (0, 0, 0)),
            pl.BlockSpec(memory_space=pl.ANY),
        ],
        out_specs=pl.BlockSpec((2, BS, S, 128),
                               lambda c, i: (0, c * NB2 + i, 0, 0)),
        out_shape=jax.ShapeDtypeStruct((2, B, S, 128), jnp.bfloat16),
        scratch_shapes=[
            pltpu.VMEM((2, BS, 2 * S, 128), jnp.float32),
            pltpu.VMEM((2, BS, 2 * S, 128), jnp.float32),
            pltpu.SemaphoreType.DMA((2,)),
        ],
        compiler_params=pltpu.CompilerParams(
            dimension_semantics=sem1),
        name="combine_relu",
    )(dp_flat, cadj, a3, brow, b1r, sab)

    xall = xh.reshape(2 * B, H1)
    w2b = W2.astype(jnp.bfloat16)
    b2r = b2.reshape(1, H2)
    w3p = jnp.pad(W3, ((0, 0), (0, 127))).astype(jnp.bfloat16)
    b3p = jnp.pad(b3.reshape(1, 1), ((0, 0), (0, 127)))

    y2 = pl.pallas_call(
        _k3_mlp,
        grid=(NC, NB3),
        in_specs=[
            pl.BlockSpec((BS3, H1), lambda c, i: (c * NB3 + i, 0)),
            pl.BlockSpec((H1, H2), lambda c, i: (0, 0)),
            pl.BlockSpec((1, H2), lambda c, i: (0, 0)),
            pl.BlockSpec((H2, 128), lambda c, i: (0, 0)),
            pl.BlockSpec((1, 128), lambda c, i: (0, 0)),
        ],
        out_specs=pl.BlockSpec((BS3, 128), lambda c, i: (c * NB3 + i, 0)),
        out_shape=jax.ShapeDtypeStruct((2 * B, 128), jnp.float32),
        compiler_params=pltpu.CompilerParams(
            dimension_semantics=sem1),
        name="mlp_tail",
    )(xall, w2b, b2r, w3p, b3p)

    return (y2[0:B, 0], y2[B:2 * B, 0])


# speedup vs baseline: 4.9532x; 1.2228x over previous
import functools

import jax
import jax.numpy as jnp
from jax.experimental import pallas as pl
from jax.experimental.pallas import tpu as pltpu


def _k1_gather_sum(tgt_s, wcat_hbm, out_ref, land, sems, *, T, DB, NBI):
    j = pl.program_id(1)
    blk = pl.program_id(0) * NBI + j
    cur = jax.lax.rem(j, 3)

    def issue(b, slot):
        base = b * DB * T
        for di in range(DB):
            for t in range(T):
                r = tgt_s[base + di * T + t]
                pltpu.make_async_copy(
                    wcat_hbm.at[r], land.at[slot, di, t], sems.at[slot]).start()

    @pl.when(j == 0)
    def _():
        issue(blk, cur)
        if NBI > 1:
            issue(blk + 1, jax.lax.rem(j + 1, 3))

    pltpu.make_async_copy(land.at[cur], land.at[cur], sems.at[cur]).wait()

    @pl.when(j + 2 < NBI)
    def _():
        issue(blk + 2, jax.lax.rem(j + 2, 3))

    for di in range(DB):
        acc = land[cur, di, 0].astype(jnp.float32)
        for t in range(1, T):
            acc = acc + land[cur, di, t].astype(jnp.float32)
        out_ref[di] = acc


def _k2_combine(dp_s, cadj_ref, a_ref, b_ref, b1_ref, sab_hbm, out_ref,
                land1, land2, sems, *, BS, S, NBI):
    j = pl.program_id(1)
    blk = pl.program_id(0) * NBI + j
    cur = jax.lax.rem(j, 3)

    def issue(b, slot):
        base = b * BS * 2
        for i in range(BS):
            d1 = dp_s[base + 2 * i]
            d2 = dp_s[base + 2 * i + 1]
            pltpu.make_async_copy(
                sab_hbm.at[d1], land1.at[slot, i], sems.at[slot]).start()
            pltpu.make_async_copy(
                sab_hbm.at[d2], land2.at[slot, i], sems.at[slot]).start()

    @pl.when(j == 0)
    def _():
        issue(blk, cur)
        if NBI > 1:
            issue(blk + 1, jax.lax.rem(j + 1, 3))

    pltpu.make_async_copy(land1.at[cur], land1.at[cur], sems.at[cur]).wait()
    pltpu.make_async_copy(land2.at[cur], land2.at[cur], sems.at[cur]).wait()

    @pl.when(j + 2 < NBI)
    def _():
        issue(blk + 2, jax.lax.rem(j + 2, 3))

    c1 = cadj_ref[:, 0:1].reshape(BS, 1, 1)
    c2 = cadj_ref[:, 1:2].reshape(BS, 1, 1)
    a = a_ref[...]
    b = b_ref[...]
    b1 = b1_ref[...]
    l1 = land1[cur]
    l2 = land2[cur]
    x1 = l1[:, :S] + l2[:, S:] + c1 * a + c2 * b + b1
    x2 = l2[:, :S] + l1[:, S:] + c2 * a + c1 * b + b1
    out_ref[0] = jnp.maximum(x1, 0.0).astype(jnp.bfloat16)
    out_ref[1] = jnp.maximum(x2, 0.0).astype(jnp.bfloat16)


def _k3_mlp(x_ref, w2_ref, b2_ref, w3_ref, b3_ref, o_ref):
    h = jnp.dot(x_ref[...], w2_ref[...], preferred_element_type=jnp.float32)
    h = jnp.maximum(h + b2_ref[...], 0.0).astype(jnp.bfloat16)
    y = jnp.dot(h, w3_ref[...], preferred_element_type=jnp.float32)
    o_ref[...] = y + b3_ref[...]


def _pick_block(n, want):
    for cand in (want, 256, 128, 64, 32, 16, 8, 4, 2, 1):
        if cand <= want and n % cand == 0:
            return cand
    return 1


def kernel(drug_pairs, drug_targets, conc, W1, b1, W2, b2, W3, b3):
    B = drug_pairs.shape[0]
    D, T = drug_targets.shape
    DIN, H1 = W1.shape
    H2 = W2.shape[1]
    P = (DIN - 2) // 2
    S = H1 // 128

    tgt = drug_targets.astype(jnp.int32)
    eq = tgt[:, :, None] == tgt[:, None, :]
    earlier = jnp.tril(jnp.ones((T, T), jnp.bool_), k=-1)
    isdup = jnp.any(eq & earlier[None], axis=2)
    tgt_a = jnp.where(isdup, P, tgt)
    ndup = jnp.sum(isdup, axis=1).astype(jnp.float32)
    dp = drug_pairs.astype(jnp.int32)
    cadj = conc.astype(jnp.float32) - ndup[dp]

    wcat = jnp.concatenate([W1[:P], W1[P + 1:2 * P + 1]], axis=1)
    ab = jnp.concatenate([W1[P], W1[2 * P + 1]])[None]
    wcat = jnp.concatenate([wcat, ab], axis=0).astype(jnp.bfloat16)
    wcat = wcat.reshape(P + 1, 2 * S, 128)

    a3 = W1[P].reshape(1, S, 128)
    brow = W1[2 * P + 1].reshape(1, S, 128)
    b1r = b1.reshape(1, S, 128)
    tgt_flat = tgt_a.reshape(D * T)
    dp_flat = dp.reshape(2 * B)

    DB = _pick_block(D, 16)
    BS = _pick_block(B, 128)
    BS3 = _pick_block(2 * B, 512)
    NC = 1
    NB1 = D // DB // NC
    NB2 = B // BS // NC
    NB3 = 2 * B // BS3 // NC
    sem1 = ("core_parallel", "arbitrary") if NC == 2 else ("arbitrary", "arbitrary")

    sab = pl.pallas_call(
        functools.partial(_k1_gather_sum, T=T, DB=DB, NBI=NB1),
        grid=(NC, NB1),
        in_specs=[
            pl.BlockSpec(memory_space=pltpu.SMEM),
            pl.BlockSpec(memory_space=pl.ANY),
        ],
        out_specs=pl.BlockSpec((DB, 2 * S, 128),
                               lambda c, i: (c * NB1 + i, 0, 0)),
        out_shape=jax.ShapeDtypeStruct((D, 2 * S, 128), jnp.float32),
        scratch_shapes=[
            pltpu.VMEM((3, DB, T, 2 * S, 128), jnp.bfloat16),
            pltpu.SemaphoreType.DMA((3,)),
        ],
        compiler_params=pltpu.CompilerParams(
            dimension_semantics=sem1),
        name="drug_gather_sum",
    )(tgt_flat, wcat)

    xh = pl.pallas_call(
        functools.partial(_k2_combine, BS=BS, S=S, NBI=NB2),
        grid=(NC, NB2),
        in_specs=[
            pl.BlockSpec(memory_space=pltpu.SMEM),
            pl.BlockSpec((BS, 2), lambda c, i: (c * NB2 + i, 0)),
            pl.BlockSpec((1, S, 128), lambda c, i: (0, 0, 0)),
            pl.BlockSpec((1, S, 128), lambda c, i: (0, 0, 0)),
            pl.BlockSpec((1, S, 128), lambda c, i: (0, 0, 0)),
            pl.BlockSpec(memory_space=pl.ANY),
        ],
        out_specs=pl.BlockSpec((2, BS, S, 128),
                               lambda c, i: (0, c * NB2 + i, 0, 0)),
        out_shape=jax.ShapeDtypeStruct((2, B, S, 128), jnp.bfloat16),
        scratch_shapes=[
            pltpu.VMEM((3, BS, 2 * S, 128), jnp.float32),
            pltpu.VMEM((3, BS, 2 * S, 128), jnp.float32),
            pltpu.SemaphoreType.DMA((3,)),
        ],
        compiler_params=pltpu.CompilerParams(
            dimension_semantics=sem1),
        name="combine_relu",
    )(dp_flat, cadj, a3, brow, b1r, sab)

    xall = xh.reshape(2 * B, H1)
    w2b = W2.astype(jnp.bfloat16)
    b2r = b2.reshape(1, H2)
    w3p = jnp.pad(W3, ((0, 0), (0, 127))).astype(jnp.bfloat16)
    b3p = jnp.pad(b3.reshape(1, 1), ((0, 0), (0, 127)))

    y2 = pl.pallas_call(
        _k3_mlp,
        grid=(NC, NB3),
        in_specs=[
            pl.BlockSpec((BS3, H1), lambda c, i: (c * NB3 + i, 0)),
            pl.BlockSpec((H1, H2), lambda c, i: (0, 0)),
            pl.BlockSpec((1, H2), lambda c, i: (0, 0)),
            pl.BlockSpec((H2, 128), lambda c, i: (0, 0)),
            pl.BlockSpec((1, 128), lambda c, i: (0, 0)),
        ],
        out_specs=pl.BlockSpec((BS3, 128), lambda c, i: (c * NB3 + i, 0)),
        out_shape=jax.ShapeDtypeStruct((2 * B, 128), jnp.float32),
        compiler_params=pltpu.CompilerParams(
            dimension_semantics=sem1),
        name="mlp_tail",
    )(xall, w2b, b2r, w3p, b3p)

    return (y2[0:B, 0], y2[B:2 * B, 0])
